# baseline (speedup 1.0000x reference)
import jax
import jax.numpy as jnp
from jax.experimental import pallas as pl
from jax.experimental.pallas import tpu as pltpu


def _mlp_body(x_ref, a_ref, c_ref, w1_ref, b1_ref, w2_ref, b2_ref, o_ref):
    x_t = x_ref[...]
    h_t = jnp.dot(a_ref[...], x_t, preferred_element_type=jnp.float32)
    h_t = jnp.maximum(h_t.astype(jnp.bfloat16) + c_ref[...], 0)
    h2_t = jnp.dot(w1_ref[...], h_t, preferred_element_type=jnp.float32)
    h2_t = jnp.maximum(h2_t.astype(jnp.bfloat16) + b1_ref[...], 0)
    s = jnp.dot(w2_ref[...], h2_t, preferred_element_type=jnp.float32)
    o_ref[...] = (s + b2_ref[0]).reshape(o_ref.shape)


def kernel(node_feats, edge_index, edge_attr, encoded_vnfs,
           W_node, b_node, W_edge, b_edge,
           W_self0, b_self0, W_neigh0, b_neigh0,
           W_self1, b_self1, W_neigh1, b_neigh1,
           W_att, b_att, W_s1, b_s1, W_s2, b_s2):
    E = edge_attr.shape[0]
    H = W_att.shape[0]

    vnf_mean = jnp.mean(encoded_vnfs, axis=0)
    A2 = W_att[:, :H] @ W_edge
    c = (W_att[:, :H] @ b_edge + W_att[:, H:] @ vnf_mean + b_att)[:, None]

    x_t = edge_attr.T.astype(jnp.bfloat16)

    BLK = 32768
    grid = pl.cdiv(E, BLK)

    out = pl.pallas_call(
        _mlp_body,
        grid=(grid,),
        in_specs=[
            pl.BlockSpec((3, BLK), lambda i: (0, i)),
            pl.BlockSpec((H, 3), lambda i: (0, 0)),
            pl.BlockSpec((H, 1), lambda i: (0, 0)),
            pl.BlockSpec((H, H), lambda i: (0, 0)),
            pl.BlockSpec((H, 1), lambda i: (0, 0)),
            pl.BlockSpec((1, H), lambda i: (0, 0)),
            pl.BlockSpec(memory_space=pltpu.SMEM),
        ],
        out_specs=pl.BlockSpec((BLK,), lambda i: (i,)),
        out_shape=jax.ShapeDtypeStruct((E,), jnp.float32),
        compiler_params=pltpu.CompilerParams(
            dimension_semantics=("parallel",),
        ),
    )(x_t, A2.astype(jnp.bfloat16), c.astype(jnp.bfloat16), W_s1.astype(jnp.bfloat16),
      b_s1[:, None].astype(jnp.bfloat16), W_s2.astype(jnp.bfloat16), b_s2)

    return out


# speedup vs baseline: 1.1003x; 1.1003x over previous
import jax
import jax.numpy as jnp
from jax.experimental import pallas as pl
from jax.experimental.pallas import tpu as pltpu


def _mlp_body(x_ref, a_ref, c_ref, w1_ref, b1_ref, w2_ref, b2_ref, o_ref):
    x_t = x_ref[...]
    h_t = jnp.dot(a_ref[...], x_t, preferred_element_type=jnp.float32)
    h_t = jnp.maximum(h_t.astype(jnp.bfloat16) + c_ref[...], 0)
    h2_t = jnp.dot(w1_ref[...], h_t, preferred_element_type=jnp.float32)
    h2_t = jnp.maximum(h2_t.astype(jnp.bfloat16) + b1_ref[...], 0)
    s = jnp.dot(w2_ref[...], h2_t, preferred_element_type=jnp.float32)
    o_ref[...] = (s + b2_ref[0]).reshape(o_ref.shape)


def kernel(node_feats, edge_index, edge_attr, encoded_vnfs,
           W_node, b_node, W_edge, b_edge,
           W_self0, b_self0, W_neigh0, b_neigh0,
           W_self1, b_self1, W_neigh1, b_neigh1,
           W_att, b_att, W_s1, b_s1, W_s2, b_s2):
    E = edge_attr.shape[0]
    H = W_att.shape[0]

    vnf_mean = jnp.mean(encoded_vnfs, axis=0)
    A2 = W_att[:, :H] @ W_edge
    c = (W_att[:, :H] @ b_edge + W_att[:, H:] @ vnf_mean + b_att)[:, None]

    x_t = edge_attr.T

    BLK = 32768
    grid = pl.cdiv(E, BLK)

    out = pl.pallas_call(
        _mlp_body,
        grid=(grid,),
        in_specs=[
            pl.BlockSpec((3, BLK), lambda i: (0, i)),
            pl.BlockSpec((H, 3), lambda i: (0, 0)),
            pl.BlockSpec((H, 1), lambda i: (0, 0)),
            pl.BlockSpec((H, H), lambda i: (0, 0)),
            pl.BlockSpec((H, 1), lambda i: (0, 0)),
            pl.BlockSpec((1, H), lambda i: (0, 0)),
            pl.BlockSpec(memory_space=pltpu.SMEM),
        ],
        out_specs=pl.BlockSpec((BLK,), lambda i: (i,)),
        out_shape=jax.ShapeDtypeStruct((E,), jnp.float32),
        compiler_params=pltpu.CompilerParams(
            dimension_semantics=("parallel",),
        ),
    )(x_t, A2, c.astype(jnp.bfloat16), W_s1.astype(jnp.bfloat16),
      b_s1[:, None].astype(jnp.bfloat16), W_s2.astype(jnp.bfloat16), b_s2)

    return out


# speedup vs baseline: 1.1117x; 1.0103x over previous
import jax
import jax.numpy as jnp
from jax.experimental import pallas as pl
from jax.experimental.pallas import tpu as pltpu


def _mlp_body(x_ref, a_ref, c_ref, w1_ref, b1_ref, w2_ref, b2_ref, o_ref):
    x_t = x_ref[...]
    h_t = jnp.dot(a_ref[...], x_t, preferred_element_type=jnp.float32)
    h_t = jnp.maximum(h_t.astype(jnp.bfloat16) + c_ref[...], 0)
    h2_t = jnp.dot(w1_ref[...], h_t, preferred_element_type=jnp.float32)
    h2_t = jnp.maximum(h2_t.astype(jnp.bfloat16) + b1_ref[...], 0)
    s = jnp.dot(w2_ref[...], h2_t, preferred_element_type=jnp.float32)
    o_ref[...] = (s + b2_ref[0]).reshape(o_ref.shape)


def kernel(node_feats, edge_index, edge_attr, encoded_vnfs,
           W_node, b_node, W_edge, b_edge,
           W_self0, b_self0, W_neigh0, b_neigh0,
           W_self1, b_self1, W_neigh1, b_neigh1,
           W_att, b_att, W_s1, b_s1, W_s2, b_s2):
    E = edge_attr.shape[0]
    H = W_att.shape[0]

    vnf_mean = jnp.mean(encoded_vnfs, axis=0)
    A2 = W_att[:, :H] @ W_edge
    c = (W_att[:, :H] @ b_edge + W_att[:, H:] @ vnf_mean + b_att)[:, None]

    x_t = edge_attr.T

    BLK = 28672
    grid = pl.cdiv(E, BLK)

    out = pl.pallas_call(
        _mlp_body,
        grid=(grid,),
        in_specs=[
            pl.BlockSpec((3, BLK), lambda i: (0, i)),
            pl.BlockSpec((H, 3), lambda i: (0, 0)),
            pl.BlockSpec((H, 1), lambda i: (0, 0)),
            pl.BlockSpec((H, H), lambda i: (0, 0)),
            pl.BlockSpec((H, 1), lambda i: (0, 0)),
            pl.BlockSpec((1, H), lambda i: (0, 0)),
            pl.BlockSpec(memory_space=pltpu.SMEM),
        ],
        out_specs=pl.BlockSpec((BLK,), lambda i: (i,)),
        out_shape=jax.ShapeDtypeStruct((E,), jnp.float32),
        compiler_params=pltpu.CompilerParams(
            dimension_semantics=("parallel",),
        ),
    )(x_t, A2, c.astype(jnp.bfloat16), W_s1.astype(jnp.bfloat16),
      b_s1[:, None].astype(jnp.bfloat16), W_s2.astype(jnp.bfloat16), b_s2)

    return out
